# Initial kernel scaffold; baseline (speedup 1.0000x reference)
import jax
import jax.numpy as jnp
from jax.experimental import pallas as pl


def kernel(x):
    raise NotImplementedError("write your pallas kernel here")


# speedup vs baseline: 3.0457x; 3.0457x over previous
import functools

import jax
import jax.numpy as jnp
from jax.experimental import pallas as pl
from jax.experimental.pallas import tpu as pltpu

_SPARSE_RATIO = 0.5
_CL = 512


def _body(x_ref, o_ref, nb_ref, sc_ref, *, K: int, L: int):
    nc = L // _CL
    rows = _CL // 128
    c = pl.program_id(1)

    @pl.when(c == 0)
    def _per_row():
        for i in range(nc):
            part = x_ref[0, pl.ds(i * _CL, _CL), :]
            s = jnp.sum(part * part, axis=-1)
            bits = jax.lax.bitcast_convert_type(jnp.sqrt(s), jnp.int32)
            nb_ref[pl.ds(i * rows, rows), :] = bits.reshape(rows, 128)

        bits2 = nb_ref[...]

        def step(i, t):
            cand = jnp.bitwise_or(t, jnp.left_shift(jnp.int32(1), 30 - i))
            cnt = jnp.sum(jnp.where(bits2 >= cand, 1, 0))
            return jnp.where(cnt >= K, cand, t)

        T = jax.lax.fori_loop(0, 31, step, jnp.int32(0))
        eq2 = bits2 == T
        g = jnp.sum(jnp.where(bits2 > T, 1, 0))
        sc_ref[0] = T
        sc_ref[1] = K - g
        riota = jax.lax.broadcasted_iota(jnp.int32, bits2.shape, 0)
        for cc in range(nc):
            sc_ref[2 + cc] = jnp.sum(
                jnp.where(jnp.logical_and(eq2, riota < cc * rows), 1, 0))

    T = sc_ref[0]
    m = sc_ref[1]
    pre = sc_ref[2 + c]

    bc = nb_ref[pl.ds(c * rows, rows), :]
    gt = bc > T
    eq = bc == T
    eqf = eq.astype(jnp.float32)
    upper = (jax.lax.broadcasted_iota(jnp.int32, (128, 128), 0)
             <= jax.lax.broadcasted_iota(jnp.int32, (128, 128), 1)
             ).astype(jnp.float32)
    pref = jnp.dot(eqf, upper, preferred_element_type=jnp.float32)
    row_tot = pref[:, 127:128]
    lstrict = (jax.lax.broadcasted_iota(jnp.int32, (rows, rows), 1)
               < jax.lax.broadcasted_iota(jnp.int32, (rows, rows), 0)
               ).astype(jnp.float32)
    off = jnp.dot(lstrict, row_tot, preferred_element_type=jnp.float32)
    cum = pref + off + pre.astype(jnp.float32)

    keep = jnp.logical_or(
        gt, jnp.logical_and(eq, cum <= m.astype(jnp.float32)))
    keepf = keep.astype(jnp.float32)

    sel = (jax.lax.broadcasted_iota(jnp.int32, (_CL, 128), 0) % 128
           == jax.lax.broadcasted_iota(jnp.int32, (_CL, 128), 1)
           ).astype(jnp.float32)
    tmp = jax.lax.dot_general(sel, keepf, (((1,), (1,)), ((), ())),
                              preferred_element_type=jnp.float32)
    ind = (jax.lax.broadcasted_iota(jnp.int32, (_CL, rows), 0) // 128
           == jax.lax.broadcasted_iota(jnp.int32, (_CL, rows), 1)
           ).astype(jnp.float32)
    keepcol = jnp.sum(tmp * ind, axis=1, keepdims=True)
    o_ref[0] = x_ref[0, pl.ds(c * _CL, _CL), :] * keepcol


def kernel(x):
    B, L, C = x.shape
    K = max(1, int(L * (1.0 - _SPARSE_RATIO)))
    nc = L // _CL

    return pl.pallas_call(
        functools.partial(_body, K=K, L=L),
        grid=(B, nc),
        in_specs=[pl.BlockSpec((1, L, C), lambda b, c: (b, 0, 0))],
        out_specs=pl.BlockSpec((1, _CL, C), lambda b, c: (b, c, 0)),
        out_shape=jax.ShapeDtypeStruct((B, L, C), x.dtype),
        scratch_shapes=[
            pltpu.VMEM((L // 128, 128), jnp.int32),
            pltpu.SMEM((2 + nc,), jnp.int32),
        ],
    )(x)


# speedup vs baseline: 3.2770x; 1.0760x over previous
import functools

import jax
import jax.numpy as jnp
from jax.experimental import pallas as pl
from jax.experimental.pallas import tpu as pltpu

_SPARSE_RATIO = 0.5
_CL = 512


def _body(x_ref, ltri_ref, o_ref, nb_ref, nbcol_ref, sc_ref, *, K: int, L: int):
    nc = L // _CL
    rows = _CL // 128
    c = pl.program_id(1)

    @pl.when(c == 0)
    def _per_row():
        for i in range(nc):
            part = x_ref[0, pl.ds(i * _CL, _CL), :]
            s = jnp.sum(part * part, axis=-1)
            bits = jax.lax.bitcast_convert_type(jnp.sqrt(s), jnp.int32)
            nbcol_ref[pl.ds(i * _CL, _CL), :] = bits[:, None]
            nb_ref[pl.ds(i * rows, rows), :] = bits.reshape(rows, 128)

        bits2 = nb_ref[...]

        def step(i, t):
            cand = jnp.bitwise_or(t, jnp.left_shift(jnp.int32(1), 30 - i))
            cnt = jnp.sum(jnp.where(bits2 >= cand, 1, 0))
            return jnp.where(cnt >= K, cand, t)

        T = jax.lax.fori_loop(0, 31, step, jnp.int32(0))
        eq2 = bits2 == T
        g = jnp.sum(jnp.where(bits2 > T, 1, 0))
        sc_ref[0] = T
        sc_ref[1] = K - g
        riota = jax.lax.broadcasted_iota(jnp.int32, bits2.shape, 0)
        for cc in range(nc):
            sc_ref[2 + cc] = jnp.sum(
                jnp.where(jnp.logical_and(eq2, riota < cc * rows), 1, 0))

    T = sc_ref[0]
    m = sc_ref[1]
    pre = sc_ref[2 + c]

    bc = nbcol_ref[pl.ds(c * _CL, _CL), :]
    gt = (bc > T).astype(jnp.float32)
    eq = (bc == T).astype(jnp.float32)
    pref = jnp.dot(ltri_ref[...], eq, preferred_element_type=jnp.float32)
    cum = pref + pre.astype(jnp.float32)
    keepcol = gt + eq * (cum <= m.astype(jnp.float32)).astype(jnp.float32)
    o_ref[0] = x_ref[0, pl.ds(c * _CL, _CL), :] * keepcol


def kernel(x):
    B, L, C = x.shape
    K = max(1, int(L * (1.0 - _SPARSE_RATIO)))
    nc = L // _CL

    ltri = jnp.tri(_CL, dtype=jnp.float32)

    return pl.pallas_call(
        functools.partial(_body, K=K, L=L),
        grid=(B, nc),
        in_specs=[
            pl.BlockSpec((1, L, C), lambda b, c: (b, 0, 0)),
            pl.BlockSpec((_CL, _CL), lambda b, c: (0, 0)),
        ],
        out_specs=pl.BlockSpec((1, _CL, C), lambda b, c: (b, c, 0)),
        out_shape=jax.ShapeDtypeStruct((B, L, C), x.dtype),
        scratch_shapes=[
            pltpu.VMEM((L // 128, 128), jnp.int32),
            pltpu.VMEM((L, 1), jnp.int32),
            pltpu.SMEM((2 + nc,), jnp.int32),
        ],
    )(x, ltri)
